# baseline (speedup 1.0000x reference)
import functools

import jax
import jax.numpy as jnp
from jax import lax
from jax.experimental import pallas as pl
from jax.experimental.pallas import tpu as pltpu
from jax.experimental.pallas import tpu_sc as plsc

_NC = 2
_NS = 16
_NW = _NC * _NS
_L = 16
_CH = 128


def _mesh():
    return plsc.VectorSubcoreMesh(core_axis_name="c", subcore_axis_name="s")


def _sc_deg(idx2, wp, zeros_n, N, iters, stride):
    @functools.partial(
        pl.kernel,
        out_type=jax.ShapeDtypeStruct((_NC, N), jnp.float32),
        mesh=_mesh(),
        compiler_params=pltpu.CompilerParams(needs_layout_passes=False),
        scratch_types=[
            pltpu.VMEM((stride, 2, _CH), jnp.int32),
            pltpu.VMEM((stride, _CH), jnp.float32),
            pltpu.VMEM_SHARED((N,), jnp.float32),
            pltpu.SemaphoreType.DMA,
        ],
    )
    def k(idx_h, wp_h, zn_h, out_h, meta_v, w_v, deg_sh, sem):
        cid = lax.axis_index("c")
        sid = lax.axis_index("s")
        wid = sid * _NC + cid
        base = wid * stride
        pltpu.sync_copy(idx_h.at[pl.ds(base, stride)], meta_v)
        pltpu.sync_copy(wp_h.at[pl.ds(base, stride)], w_v)

        @pl.when(sid == 0)
        def _():
            pltpu.sync_copy(zn_h, deg_sh)

        plsc.subcore_barrier()

        def cbody(i, carry):
            descs = []
            for b in range(4):
                c = i * 4 + b
                descs.append(pltpu.async_copy(
                    w_v.at[c], deg_sh.at[meta_v.at[c, 1]], sem, add=True))
            for d in descs:
                d.wait()
            return carry

        lax.fori_loop(0, iters // 4, cbody, 0)
        plsc.subcore_barrier()

        @pl.when(sid == 0)
        def _():
            pltpu.sync_copy(deg_sh, out_h.at[cid])

    return k(idx2, wp, zeros_n)


def _sc_spmm(idx2, wp, tables, zeros_f, N, iters, stride, D):
    nh = len(tables)
    nv = D // _L

    @functools.partial(
        pl.kernel,
        out_type=jax.ShapeDtypeStruct((nh, _NC, N, D), jnp.float32),
        mesh=_mesh(),
        compiler_params=pltpu.CompilerParams(needs_layout_passes=False),
        scratch_types=[
            pltpu.VMEM((stride, 2, _CH), jnp.int32),
            pltpu.VMEM((stride, _CH), jnp.float32),
            pltpu.VMEM((_CH, D), jnp.float32),
            pltpu.VMEM((_CH, D), jnp.float32),
            pltpu.VMEM_SHARED((N, D), jnp.float32),
            pltpu.SemaphoreType.DMA,
            pltpu.SemaphoreType.DMA,
            pltpu.SemaphoreType.DMA,
            pltpu.SemaphoreType.DMA,
        ],
    )
    def k(idx_h, wp_h, *rest):
        tabs = rest[:nh]
        zf_h = rest[nh]
        out_h = rest[nh + 1]
        (meta_v, w_v, rows0, rows1, acc_sh,
         sg0, sg1, ss0, ss1) = rest[nh + 2:]
        rows = (rows0, rows1)
        sg = (sg0, sg1)
        ss = (ss0, ss1)
        cid = lax.axis_index("c")
        sid = lax.axis_index("s")
        wid = sid * _NC + cid
        base = wid * stride
        pltpu.sync_copy(idx_h.at[pl.ds(base, stride)], meta_v)
        pltpu.sync_copy(wp_h.at[pl.ds(base, stride)], w_v)

        def scale(c, rows_v):
            def sbody(g, c2):
                cvec = w_v[c, pl.ds(g * _L, _L)]
                for t in range(_L):
                    ce = cvec[t]
                    e = g * _L + t
                    for v in range(nv):
                        sv = pl.ds(v * _L, _L)
                        rows_v[e, sv] = rows_v[e, sv] * ce
                return c2

            lax.fori_loop(0, _CH // _L, sbody, 0)

        for h in range(nh):
            tab = tabs[h]

            @pl.when(sid == 0)
            def _():
                pltpu.sync_copy(zf_h, acc_sh)

            plsc.subcore_barrier()
            pltpu.async_copy(tab.at[meta_v.at[0, 0]], rows[0], sg[0])

            def cbody(i, carry):
                for b in range(2):
                    c = i * 2 + b
                    cur = rows[b]
                    oth = rows[1 - b]
                    pltpu.make_async_copy(
                        tab.at[meta_v.at[c, 0]], cur, sg[b]).wait()
                    scale(c, cur)
                    cm1 = jnp.maximum(c - 1, 0)

                    @pl.when(c >= 1)
                    def _():
                        pltpu.make_async_copy(
                            oth, acc_sh.at[meta_v.at[cm1, 1]],
                            ss[1 - b]).wait()

                    nxt = jnp.minimum(c + 1, iters - 1)
                    pltpu.async_copy(
                        tab.at[meta_v.at[nxt, 0]], oth, sg[1 - b])
                    pltpu.async_copy(
                        cur, acc_sh.at[meta_v.at[c, 1]], ss[b], add=True)
                return carry

            lax.fori_loop(0, iters // 2, cbody, 0)
            last = iters - 1
            pltpu.make_async_copy(
                tab.at[meta_v.at[last, 0]], rows[0], sg[0]).wait()
            pltpu.make_async_copy(
                rows[1], acc_sh.at[meta_v.at[last, 1]], ss[1]).wait()
            plsc.subcore_barrier()

            @pl.when(sid == 0)
            def _():
                pltpu.sync_copy(acc_sh, out_h.at[h, cid])

            plsc.subcore_barrier()

    return k(idx2, wp, *tables, zeros_f)


def _sc_edge_mlp(idx2, p, q, zeros_n, N, iters, stride, nch):
    @functools.partial(
        pl.kernel,
        out_type=[
            jax.ShapeDtypeStruct((_NW * stride, _CH), jnp.float32),
            jax.ShapeDtypeStruct((_NC, N), jnp.float32),
        ],
        mesh=_mesh(),
        compiler_params=pltpu.CompilerParams(needs_layout_passes=False),
        scratch_types=[
            pltpu.VMEM((N,), jnp.float32),
            pltpu.VMEM((N,), jnp.float32),
            pltpu.VMEM((stride, 2, _CH), jnp.int32),
            pltpu.VMEM((stride, _CH), jnp.float32),
            pltpu.VMEM_SHARED((N,), jnp.float32),
            pltpu.SemaphoreType.DMA,
        ],
    )
    def k(idx_h, p_h, q_h, zn_h, ea_h, dout_h, p_v, q_v, meta_v, ea_v,
          deg_sh, sem):
        cid = lax.axis_index("c")
        sid = lax.axis_index("s")
        wid = sid * _NC + cid
        base = wid * stride
        pltpu.sync_copy(idx_h.at[pl.ds(base, stride)], meta_v)
        pltpu.sync_copy(p_h, p_v)
        pltpu.sync_copy(q_h, q_v)

        @pl.when(sid == 0)
        def _():
            pltpu.sync_copy(zn_h, deg_sh)

        plsc.subcore_barrier()

        def cbody(i, carry):
            descs = []
            for b in range(2):
                c = i * 2 + b
                live = (wid * iters + c) < nch
                for j in range(_CH // _L):
                    sl = pl.ds(j * _L, _L)
                    ps = plsc.load_gather(p_v, [meta_v[c, 0, sl]])
                    qd = plsc.load_gather(q_v, [meta_v[c, 1, sl]])
                    ea_v[c, sl] = jnp.where(
                        live, jnp.maximum(ps + qd, 0.0), 0.0)
                descs.append(pltpu.async_copy(
                    ea_v.at[c], deg_sh.at[meta_v.at[c, 1]], sem, add=True))
            for d in descs:
                d.wait()
            return carry

        lax.fori_loop(0, iters // 2, cbody, 0)
        pltpu.sync_copy(ea_v, ea_h.at[pl.ds(base, stride)])
        plsc.subcore_barrier()

        @pl.when(sid == 0)
        def _():
            pltpu.sync_copy(deg_sh, dout_h.at[cid])

    return k(idx2, p, q, zeros_n)


def kernel(node_attr, edge_index, edge_attr, W1, b1, W2, b2, Wm, bm):
    N, Din = node_attr.shape
    E = edge_index.shape[1]
    H1 = W1.shape[1]
    H2 = W2.shape[1]
    Hh = H1 // 2
    f32 = jnp.float32

    src = edge_index[0]
    dst = edge_index[1]
    ew = edge_attr.reshape(-1)
    zeros_n = jnp.zeros((N,), f32)
    zeros_f = jnp.zeros((N, Hh), f32)

    nch = E // _CH
    iters = -(-nch // _NW)
    iters = -(-iters // 4) * 4
    stride = -(-iters // 8) * 8
    slots = iters * _NW
    pad = slots - nch

    def _lay(x2d):
        x = x2d.reshape(_NW, iters, _CH)
        x = jnp.pad(x, ((0, 0), (0, stride - iters), (0, 0)))
        return x.reshape(_NW * stride, _CH)

    pad_idx = (jnp.arange(pad * _CH, dtype=src.dtype) % N).reshape(pad, _CH)
    srcp = _lay(jnp.concatenate([src.reshape(nch, _CH), pad_idx]))
    dstp = _lay(jnp.concatenate([dst.reshape(nch, _CH), pad_idx]))
    idx2 = jnp.stack([srcp, dstp], axis=1)
    wp1 = _lay(jnp.concatenate(
        [ew.reshape(nch, _CH), jnp.zeros((pad, _CH), f32)]))

    bm_grid = N // 1000
    BM = N // bm_grid
    nspec = pl.BlockSpec((BM, 1), lambda i: (i, 0))
    fspec = pl.BlockSpec((BM, Hh), lambda i: (i, 0))

    deg1 = _sc_deg(idx2, wp1, zeros_n, N, iters, stride)
    d1c = deg1.reshape(_NC, N, 1)

    def tc1(d0_ref, d1_ref, x_ref, w_ref, oa_ref, ob_ref):
        dinv = lax.rsqrt(1.0 + d0_ref[...] + d1_ref[...])
        hseg = jnp.dot(x_ref[...], w_ref[...], preferred_element_type=f32)
        hseg = dinv * hseg
        oa_ref[...] = hseg[:, :Hh]
        ob_ref[...] = hseg[:, Hh:]

    h1a, h1b = pl.pallas_call(
        tc1,
        grid=(bm_grid,),
        in_specs=[
            nspec, nspec,
            pl.BlockSpec((BM, Din), lambda i: (i, 0)),
            pl.BlockSpec((Din, H1), lambda i: (0, 0)),
        ],
        out_specs=[fspec, fspec],
        out_shape=[jax.ShapeDtypeStruct((N, Hh), f32)] * 2,
    )(d1c[0], d1c[1], node_attr, W1)

    acc1 = _sc_spmm(idx2, wp1, [h1a, h1b], zeros_f, N, iters, stride, Hh)

    def tc2(d0_ref, d1_ref, aa0_ref, aa1_ref, ab0_ref, ab1_ref, ha_ref,
            hb_ref, b1_ref, wm_ref, bm_ref, w2_ref, p_ref, q_ref, h2_ref):
        dinv = lax.rsqrt(1.0 + d0_ref[...] + d1_ref[...])
        suma = aa0_ref[...] + aa1_ref[...] + ha_ref[...]
        sumb = ab0_ref[...] + ab1_ref[...] + hb_ref[...]
        outa = jnp.maximum(dinv * suma + b1_ref[..., :Hh], 0.0)
        outb = jnp.maximum(dinv * sumb + b1_ref[..., Hh:], 0.0)
        out1 = jnp.concatenate([outa, outb], axis=1)
        wm = wm_ref[...]
        p_ref[...] = jnp.dot(out1, wm[:H1], preferred_element_type=f32) \
            + bm_ref[...]
        q_ref[...] = jnp.dot(out1, wm[H1:], preferred_element_type=f32)
        h2_ref[...] = jnp.dot(out1, w2_ref[...], preferred_element_type=f32)

    p, q, h2 = pl.pallas_call(
        tc2,
        grid=(bm_grid,),
        in_specs=[
            nspec, nspec, fspec, fspec, fspec, fspec, fspec, fspec,
            pl.BlockSpec((1, H1), lambda i: (0, 0)),
            pl.BlockSpec((2 * H1, 1), lambda i: (0, 0)),
            pl.BlockSpec((1, 1), lambda i: (0, 0)),
            pl.BlockSpec((H1, H2), lambda i: (0, 0)),
        ],
        out_specs=[nspec, nspec, fspec],
        out_shape=[
            jax.ShapeDtypeStruct((N, 1), f32),
            jax.ShapeDtypeStruct((N, 1), f32),
            jax.ShapeDtypeStruct((N, H2), f32),
        ],
    )(d1c[0], d1c[1], acc1[0, 0], acc1[0, 1], acc1[1, 0], acc1[1, 1],
      h1a, h1b, b1.reshape(1, H1), Wm, bm.reshape(1, 1), W2)

    wp2, deg2 = _sc_edge_mlp(idx2, p.reshape(N), q.reshape(N), zeros_n,
                             N, iters, stride, nch)
    d2c = deg2.reshape(_NC, N, 1)

    def tcd2(d0_ref, d1_ref, h2_ref, o_ref):
        dinv = lax.rsqrt(1.0 + d0_ref[...] + d1_ref[...])
        o_ref[...] = dinv * h2_ref[...]

    h2s = pl.pallas_call(
        tcd2,
        grid=(bm_grid,),
        in_specs=[nspec, nspec, fspec],
        out_specs=fspec,
        out_shape=jax.ShapeDtypeStruct((N, H2), f32),
    )(d2c[0], d2c[1], h2)

    acc2 = _sc_spmm(idx2, wp2, [h2s], zeros_f, N, iters, stride, H2)

    def tc3(d0_ref, d1_ref, a0_ref, a1_ref, h2_ref, b2_ref, o_ref):
        dinv = lax.rsqrt(1.0 + d0_ref[...] + d1_ref[...])
        s = a0_ref[...] + a1_ref[...] + h2_ref[...]
        o_ref[...] = dinv * s + b2_ref[...]

    out = pl.pallas_call(
        tc3,
        grid=(bm_grid,),
        in_specs=[
            nspec, nspec, fspec, fspec, fspec,
            pl.BlockSpec((1, H2), lambda i: (0, 0)),
        ],
        out_specs=fspec,
        out_shape=jax.ShapeDtypeStruct((N, H2), f32),
    )(d2c[0], d2c[1], acc2[0, 0], acc2[0, 1], h2s, b2.reshape(1, H2))
    return out


# speedup vs baseline: 1.1608x; 1.1608x over previous
import functools

import jax
import jax.numpy as jnp
from jax import lax
from jax.experimental import pallas as pl
from jax.experimental.pallas import tpu as pltpu
from jax.experimental.pallas import tpu_sc as plsc

_NC = 2
_NS = 16
_NW = _NC * _NS
_L = 16
_CH = 128


def _mesh():
    return plsc.VectorSubcoreMesh(core_axis_name="c", subcore_axis_name="s")


def _sc_deg(idx2, wp, zeros_n, N, iters, stride):
    @functools.partial(
        pl.kernel,
        out_type=jax.ShapeDtypeStruct((_NC, N), jnp.float32),
        mesh=_mesh(),
        compiler_params=pltpu.CompilerParams(needs_layout_passes=False),
        scratch_types=[
            pltpu.VMEM((stride, 2, _CH), jnp.int32),
            pltpu.VMEM((stride, _CH), jnp.float32),
            pltpu.VMEM_SHARED((N,), jnp.float32),
            pltpu.SemaphoreType.DMA,
        ],
    )
    def k(idx_h, wp_h, zn_h, out_h, meta_v, w_v, deg_sh, sem):
        cid = lax.axis_index("c")
        sid = lax.axis_index("s")
        wid = sid * _NC + cid
        base = wid * stride
        pltpu.sync_copy(idx_h.at[pl.ds(base, stride)], meta_v)
        pltpu.sync_copy(wp_h.at[pl.ds(base, stride)], w_v)

        @pl.when(sid == 0)
        def _():
            pltpu.sync_copy(zn_h, deg_sh)

        plsc.subcore_barrier()

        def cbody(i, carry):
            descs = []
            for b in range(4):
                c = i * 4 + b
                descs.append(pltpu.async_copy(
                    w_v.at[c], deg_sh.at[meta_v.at[c, 1]], sem, add=True))
            for d in descs:
                d.wait()
            return carry

        lax.fori_loop(0, iters // 4, cbody, 0)
        plsc.subcore_barrier()

        @pl.when(sid == 0)
        def _():
            pltpu.sync_copy(deg_sh, out_h.at[cid])

    return k(idx2, wp, zeros_n)


def _sc_spmm(idx2, wp, tables, zeros_f, N, iters, stride, D):
    nh = len(tables)
    nv = D // _L

    @functools.partial(
        pl.kernel,
        out_type=jax.ShapeDtypeStruct((nh, _NC, N, D), jnp.float32),
        mesh=_mesh(),
        compiler_params=pltpu.CompilerParams(needs_layout_passes=False),
        scratch_types=[
            pltpu.VMEM((stride, 2, _CH), jnp.int32),
            pltpu.VMEM((stride, _CH), jnp.float32),
            pltpu.VMEM((_CH, D), jnp.float32),
            pltpu.VMEM((_CH, D), jnp.float32),
            pltpu.VMEM_SHARED((N, D), jnp.float32),
            pltpu.SemaphoreType.DMA,
            pltpu.SemaphoreType.DMA,
            pltpu.SemaphoreType.DMA,
            pltpu.SemaphoreType.DMA,
        ],
    )
    def k(idx_h, wp_h, *rest):
        tabs = rest[:nh]
        zf_h = rest[nh]
        out_h = rest[nh + 1]
        (meta_v, w_v, rows0, rows1, acc_sh,
         sg0, sg1, ss0, ss1) = rest[nh + 2:]
        rows = (rows0, rows1)
        sg = (sg0, sg1)
        ss = (ss0, ss1)
        cid = lax.axis_index("c")
        sid = lax.axis_index("s")
        wid = sid * _NC + cid
        base = wid * stride
        pltpu.sync_copy(idx_h.at[pl.ds(base, stride)], meta_v)
        pltpu.sync_copy(wp_h.at[pl.ds(base, stride)], w_v)

        def scale(c, rows_v):
            def sbody(g, c2):
                cvec = w_v[c, pl.ds(g * _L, _L)]
                for t in range(_L):
                    ce = cvec[t]
                    e = g * _L + t
                    for v in range(nv):
                        sv = pl.ds(v * _L, _L)
                        rows_v[e, sv] = rows_v[e, sv] * ce
                return c2

            lax.fori_loop(0, _CH // _L, sbody, 0)

        for h in range(nh):
            tab = tabs[h]

            @pl.when(sid == 0)
            def _():
                pltpu.sync_copy(zf_h, acc_sh)

            plsc.subcore_barrier()
            pltpu.async_copy(tab.at[meta_v.at[0, 0]], rows[0], sg[0]).wait()

            def cbody(i, carry):
                for b in range(2):
                    c = i * 2 + b
                    nxt = jnp.minimum(c + 1, iters - 1)
                    g = pltpu.async_copy(
                        tab.at[meta_v.at[nxt, 0]], rows[1 - b], sg[1 - b])
                    scale(c, rows[b])
                    pltpu.sync_copy(
                        rows[b], acc_sh.at[meta_v.at[c, 1]], add=True)
                    g.wait()
                return carry

            lax.fori_loop(0, iters // 2, cbody, 0)
            plsc.subcore_barrier()

            @pl.when(sid == 0)
            def _():
                pltpu.sync_copy(acc_sh, out_h.at[h, cid])

            plsc.subcore_barrier()

    return k(idx2, wp, *tables, zeros_f)


def _sc_edge_mlp(idx2, p, q, zeros_n, N, iters, stride, nch):
    @functools.partial(
        pl.kernel,
        out_type=[
            jax.ShapeDtypeStruct((_NW * stride, _CH), jnp.float32),
            jax.ShapeDtypeStruct((_NC, N), jnp.float32),
        ],
        mesh=_mesh(),
        compiler_params=pltpu.CompilerParams(needs_layout_passes=False),
        scratch_types=[
            pltpu.VMEM((N,), jnp.float32),
            pltpu.VMEM((N,), jnp.float32),
            pltpu.VMEM((stride, 2, _CH), jnp.int32),
            pltpu.VMEM((stride, _CH), jnp.float32),
            pltpu.VMEM_SHARED((N,), jnp.float32),
            pltpu.SemaphoreType.DMA,
        ],
    )
    def k(idx_h, p_h, q_h, zn_h, ea_h, dout_h, p_v, q_v, meta_v, ea_v,
          deg_sh, sem):
        cid = lax.axis_index("c")
        sid = lax.axis_index("s")
        wid = sid * _NC + cid
        base = wid * stride
        pltpu.sync_copy(idx_h.at[pl.ds(base, stride)], meta_v)
        pltpu.sync_copy(p_h, p_v)
        pltpu.sync_copy(q_h, q_v)

        @pl.when(sid == 0)
        def _():
            pltpu.sync_copy(zn_h, deg_sh)

        plsc.subcore_barrier()

        def cbody(i, carry):
            descs = []
            for b in range(2):
                c = i * 2 + b
                live = (wid * iters + c) < nch
                for j in range(_CH // _L):
                    sl = pl.ds(j * _L, _L)
                    ps = plsc.load_gather(p_v, [meta_v[c, 0, sl]])
                    qd = plsc.load_gather(q_v, [meta_v[c, 1, sl]])
                    ea_v[c, sl] = jnp.where(
                        live, jnp.maximum(ps + qd, 0.0), 0.0)
                descs.append(pltpu.async_copy(
                    ea_v.at[c], deg_sh.at[meta_v.at[c, 1]], sem, add=True))
            for d in descs:
                d.wait()
            return carry

        lax.fori_loop(0, iters // 2, cbody, 0)
        pltpu.sync_copy(ea_v, ea_h.at[pl.ds(base, stride)])
        plsc.subcore_barrier()

        @pl.when(sid == 0)
        def _():
            pltpu.sync_copy(deg_sh, dout_h.at[cid])

    return k(idx2, p, q, zeros_n)


def kernel(node_attr, edge_index, edge_attr, W1, b1, W2, b2, Wm, bm):
    N, Din = node_attr.shape
    E = edge_index.shape[1]
    H1 = W1.shape[1]
    H2 = W2.shape[1]
    Hh = H1 // 2
    f32 = jnp.float32

    src = edge_index[0]
    dst = edge_index[1]
    ew = edge_attr.reshape(-1)
    zeros_n = jnp.zeros((N,), f32)
    zeros_f = jnp.zeros((N, Hh), f32)

    nch = E // _CH
    iters = -(-nch // _NW)
    iters = -(-iters // 4) * 4
    stride = -(-iters // 8) * 8
    slots = iters * _NW
    pad = slots - nch

    def _lay(x2d):
        x = x2d.reshape(_NW, iters, _CH)
        x = jnp.pad(x, ((0, 0), (0, stride - iters), (0, 0)))
        return x.reshape(_NW * stride, _CH)

    pad_idx = (jnp.arange(pad * _CH, dtype=src.dtype) % N).reshape(pad, _CH)
    srcp = _lay(jnp.concatenate([src.reshape(nch, _CH), pad_idx]))
    dstp = _lay(jnp.concatenate([dst.reshape(nch, _CH), pad_idx]))
    idx2 = jnp.stack([srcp, dstp], axis=1)
    wp1 = _lay(jnp.concatenate(
        [ew.reshape(nch, _CH), jnp.zeros((pad, _CH), f32)]))

    bm_grid = N // 1000
    BM = N // bm_grid
    nspec = pl.BlockSpec((BM, 1), lambda i: (i, 0))
    fspec = pl.BlockSpec((BM, Hh), lambda i: (i, 0))

    deg1 = _sc_deg(idx2, wp1, zeros_n, N, iters, stride)
    d1c = deg1.reshape(_NC, N, 1)

    def tc1(d0_ref, d1_ref, x_ref, w_ref, oa_ref, ob_ref):
        dinv = lax.rsqrt(1.0 + d0_ref[...] + d1_ref[...])
        hseg = jnp.dot(x_ref[...], w_ref[...], preferred_element_type=f32)
        hseg = dinv * hseg
        oa_ref[...] = hseg[:, :Hh]
        ob_ref[...] = hseg[:, Hh:]

    h1a, h1b = pl.pallas_call(
        tc1,
        grid=(bm_grid,),
        in_specs=[
            nspec, nspec,
            pl.BlockSpec((BM, Din), lambda i: (i, 0)),
            pl.BlockSpec((Din, H1), lambda i: (0, 0)),
        ],
        out_specs=[fspec, fspec],
        out_shape=[jax.ShapeDtypeStruct((N, Hh), f32)] * 2,
    )(d1c[0], d1c[1], node_attr, W1)

    acc1 = _sc_spmm(idx2, wp1, [h1a, h1b], zeros_f, N, iters, stride, Hh)

    def tc2(d0_ref, d1_ref, aa0_ref, aa1_ref, ab0_ref, ab1_ref, ha_ref,
            hb_ref, b1_ref, wm_ref, bm_ref, w2_ref, p_ref, q_ref, h2_ref):
        dinv = lax.rsqrt(1.0 + d0_ref[...] + d1_ref[...])
        suma = aa0_ref[...] + aa1_ref[...] + ha_ref[...]
        sumb = ab0_ref[...] + ab1_ref[...] + hb_ref[...]
        outa = jnp.maximum(dinv * suma + b1_ref[..., :Hh], 0.0)
        outb = jnp.maximum(dinv * sumb + b1_ref[..., Hh:], 0.0)
        out1 = jnp.concatenate([outa, outb], axis=1)
        wm = wm_ref[...]
        p_ref[...] = jnp.dot(out1, wm[:H1], preferred_element_type=f32) \
            + bm_ref[...]
        q_ref[...] = jnp.dot(out1, wm[H1:], preferred_element_type=f32)
        h2_ref[...] = jnp.dot(out1, w2_ref[...], preferred_element_type=f32)

    p, q, h2 = pl.pallas_call(
        tc2,
        grid=(bm_grid,),
        in_specs=[
            nspec, nspec, fspec, fspec, fspec, fspec, fspec, fspec,
            pl.BlockSpec((1, H1), lambda i: (0, 0)),
            pl.BlockSpec((2 * H1, 1), lambda i: (0, 0)),
            pl.BlockSpec((1, 1), lambda i: (0, 0)),
            pl.BlockSpec((H1, H2), lambda i: (0, 0)),
        ],
        out_specs=[nspec, nspec, fspec],
        out_shape=[
            jax.ShapeDtypeStruct((N, 1), f32),
            jax.ShapeDtypeStruct((N, 1), f32),
            jax.ShapeDtypeStruct((N, H2), f32),
        ],
    )(d1c[0], d1c[1], acc1[0, 0], acc1[0, 1], acc1[1, 0], acc1[1, 1],
      h1a, h1b, b1.reshape(1, H1), Wm, bm.reshape(1, 1), W2)

    wp2, deg2 = _sc_edge_mlp(idx2, p.reshape(N), q.reshape(N), zeros_n,
                             N, iters, stride, nch)
    d2c = deg2.reshape(_NC, N, 1)

    def tcd2(d0_ref, d1_ref, h2_ref, o_ref):
        dinv = lax.rsqrt(1.0 + d0_ref[...] + d1_ref[...])
        o_ref[...] = dinv * h2_ref[...]

    h2s = pl.pallas_call(
        tcd2,
        grid=(bm_grid,),
        in_specs=[nspec, nspec, fspec],
        out_specs=fspec,
        out_shape=jax.ShapeDtypeStruct((N, H2), f32),
    )(d2c[0], d2c[1], h2)

    acc2 = _sc_spmm(idx2, wp2, [h2s], zeros_f, N, iters, stride, H2)

    def tc3(d0_ref, d1_ref, a0_ref, a1_ref, h2_ref, b2_ref, o_ref):
        dinv = lax.rsqrt(1.0 + d0_ref[...] + d1_ref[...])
        s = a0_ref[...] + a1_ref[...] + h2_ref[...]
        o_ref[...] = dinv * s + b2_ref[...]

    out = pl.pallas_call(
        tc3,
        grid=(bm_grid,),
        in_specs=[
            nspec, nspec, fspec, fspec, fspec,
            pl.BlockSpec((1, H2), lambda i: (0, 0)),
        ],
        out_specs=fspec,
        out_shape=jax.ShapeDtypeStruct((N, H2), f32),
    )(d2c[0], d2c[1], acc2[0, 0], acc2[0, 1], h2s, b2.reshape(1, H2))
    return out


# speedup vs baseline: 1.1617x; 1.0007x over previous
import functools

import jax
import jax.numpy as jnp
from jax import lax
from jax.experimental import pallas as pl
from jax.experimental.pallas import tpu as pltpu
from jax.experimental.pallas import tpu_sc as plsc

_NC = 2
_NS = 16
_NW = _NC * _NS
_L = 16
_CH = 128


def _mesh():
    return plsc.VectorSubcoreMesh(core_axis_name="c", subcore_axis_name="s")


def _sc_deg(idx2, wp, zeros_n, N, iters, stride):
    @functools.partial(
        pl.kernel,
        out_type=jax.ShapeDtypeStruct((_NC, N), jnp.float32),
        mesh=_mesh(),
        compiler_params=pltpu.CompilerParams(needs_layout_passes=False),
        scratch_types=[
            pltpu.VMEM((stride, 2, _CH), jnp.int32),
            pltpu.VMEM((stride, _CH), jnp.float32),
            pltpu.VMEM_SHARED((N,), jnp.float32),
            pltpu.SemaphoreType.DMA,
        ],
    )
    def k(idx_h, wp_h, zn_h, out_h, meta_v, w_v, deg_sh, sem):
        cid = lax.axis_index("c")
        sid = lax.axis_index("s")
        wid = sid * _NC + cid
        base = wid * stride
        pltpu.sync_copy(idx_h.at[pl.ds(base, stride)], meta_v)
        pltpu.sync_copy(wp_h.at[pl.ds(base, stride)], w_v)

        @pl.when(sid == 0)
        def _():
            pltpu.sync_copy(zn_h, deg_sh)

        plsc.subcore_barrier()

        def cbody(i, carry):
            descs = []
            for b in range(4):
                c = i * 4 + b
                descs.append(pltpu.async_copy(
                    w_v.at[c], deg_sh.at[meta_v.at[c, 1]], sem, add=True))
            for d in descs:
                d.wait()
            return carry

        lax.fori_loop(0, iters // 4, cbody, 0)
        plsc.subcore_barrier()

        @pl.when(sid == 0)
        def _():
            pltpu.sync_copy(deg_sh, out_h.at[cid])

    return k(idx2, wp, zeros_n)


def _sc_spmm(idx2, wp, tables, zeros_f, N, iters, stride, D):
    nh = len(tables)
    nv = D // _L

    @functools.partial(
        pl.kernel,
        out_type=jax.ShapeDtypeStruct((nh, _NC, N, D), jnp.float32),
        mesh=_mesh(),
        compiler_params=pltpu.CompilerParams(needs_layout_passes=False),
        scratch_types=[
            pltpu.VMEM((stride, 2, _CH), jnp.int32),
            pltpu.VMEM((stride, _CH), jnp.float32),
            pltpu.VMEM((_CH, D), jnp.float32),
            pltpu.VMEM((_CH, D), jnp.float32),
            pltpu.VMEM_SHARED((N, D), jnp.float32),
            pltpu.SemaphoreType.DMA,
            pltpu.SemaphoreType.DMA,
            pltpu.SemaphoreType.DMA,
            pltpu.SemaphoreType.DMA,
        ],
    )
    def k(idx_h, wp_h, *rest):
        tabs = rest[:nh]
        zf_h = rest[nh]
        out_h = rest[nh + 1]
        (meta_v, w_v, rows0, rows1, acc_sh,
         sg0, sg1, ss0, ss1) = rest[nh + 2:]
        rows = (rows0, rows1)
        sg = (sg0, sg1)
        ss = (ss0, ss1)
        cid = lax.axis_index("c")
        sid = lax.axis_index("s")
        wid = sid * _NC + cid
        base = wid * stride
        pltpu.sync_copy(idx_h.at[pl.ds(base, stride)], meta_v)
        pltpu.sync_copy(wp_h.at[pl.ds(base, stride)], w_v)

        def scale(c, rows_v):
            def sbody(g):
                cvec = w_v[c, pl.ds(g * _L, _L)]
                for t in range(_L):
                    ce = cvec[t]
                    e = g * _L + t
                    for v in range(nv):
                        sv = pl.ds(v * _L, _L)
                        rows_v[e, sv] = rows_v[e, sv] * ce

            plsc.parallel_loop(0, _CH // _L)(sbody)

        for h in range(nh):
            tab = tabs[h]

            @pl.when(sid == 0)
            def _():
                pltpu.sync_copy(zf_h, acc_sh)

            plsc.subcore_barrier()
            pltpu.async_copy(tab.at[meta_v.at[0, 0]], rows[0], sg[0]).wait()

            def cbody(i, carry):
                for b in range(2):
                    c = i * 2 + b
                    nxt = jnp.minimum(c + 1, iters - 1)
                    g = pltpu.async_copy(
                        tab.at[meta_v.at[nxt, 0]], rows[1 - b], sg[1 - b])
                    scale(c, rows[b])
                    pltpu.sync_copy(
                        rows[b], acc_sh.at[meta_v.at[c, 1]], add=True)
                    g.wait()
                return carry

            lax.fori_loop(0, iters // 2, cbody, 0)
            plsc.subcore_barrier()

            @pl.when(sid == 0)
            def _():
                pltpu.sync_copy(acc_sh, out_h.at[h, cid])

            plsc.subcore_barrier()

    return k(idx2, wp, *tables, zeros_f)


def _sc_edge_mlp(idx2, p, q, zeros_n, N, iters, stride, nch):
    @functools.partial(
        pl.kernel,
        out_type=[
            jax.ShapeDtypeStruct((_NW * stride, _CH), jnp.float32),
            jax.ShapeDtypeStruct((_NC, N), jnp.float32),
        ],
        mesh=_mesh(),
        compiler_params=pltpu.CompilerParams(needs_layout_passes=False),
        scratch_types=[
            pltpu.VMEM((N,), jnp.float32),
            pltpu.VMEM((N,), jnp.float32),
            pltpu.VMEM((stride, 2, _CH), jnp.int32),
            pltpu.VMEM((stride, _CH), jnp.float32),
            pltpu.VMEM_SHARED((N,), jnp.float32),
            pltpu.SemaphoreType.DMA,
        ],
    )
    def k(idx_h, p_h, q_h, zn_h, ea_h, dout_h, p_v, q_v, meta_v, ea_v,
          deg_sh, sem):
        cid = lax.axis_index("c")
        sid = lax.axis_index("s")
        wid = sid * _NC + cid
        base = wid * stride
        pltpu.sync_copy(idx_h.at[pl.ds(base, stride)], meta_v)
        pltpu.sync_copy(p_h, p_v)
        pltpu.sync_copy(q_h, q_v)

        @pl.when(sid == 0)
        def _():
            pltpu.sync_copy(zn_h, deg_sh)

        plsc.subcore_barrier()

        def cbody(i, carry):
            descs = []
            for b in range(2):
                c = i * 2 + b
                live = (wid * iters + c) < nch
                for j in range(_CH // _L):
                    sl = pl.ds(j * _L, _L)
                    ps = plsc.load_gather(p_v, [meta_v[c, 0, sl]])
                    qd = plsc.load_gather(q_v, [meta_v[c, 1, sl]])
                    ea_v[c, sl] = jnp.where(
                        live, jnp.maximum(ps + qd, 0.0), 0.0)
                descs.append(pltpu.async_copy(
                    ea_v.at[c], deg_sh.at[meta_v.at[c, 1]], sem, add=True))
            for d in descs:
                d.wait()
            return carry

        lax.fori_loop(0, iters // 2, cbody, 0)
        pltpu.sync_copy(ea_v, ea_h.at[pl.ds(base, stride)])
        plsc.subcore_barrier()

        @pl.when(sid == 0)
        def _():
            pltpu.sync_copy(deg_sh, dout_h.at[cid])

    return k(idx2, p, q, zeros_n)


def kernel(node_attr, edge_index, edge_attr, W1, b1, W2, b2, Wm, bm):
    N, Din = node_attr.shape
    E = edge_index.shape[1]
    H1 = W1.shape[1]
    H2 = W2.shape[1]
    Hh = H1 // 2
    f32 = jnp.float32

    src = edge_index[0]
    dst = edge_index[1]
    ew = edge_attr.reshape(-1)
    zeros_n = jnp.zeros((N,), f32)
    zeros_f = jnp.zeros((N, Hh), f32)

    nch = E // _CH
    iters = -(-nch // _NW)
    iters = -(-iters // 4) * 4
    stride = -(-iters // 8) * 8
    slots = iters * _NW
    pad = slots - nch

    def _lay(x2d):
        x = x2d.reshape(_NW, iters, _CH)
        x = jnp.pad(x, ((0, 0), (0, stride - iters), (0, 0)))
        return x.reshape(_NW * stride, _CH)

    pad_idx = (jnp.arange(pad * _CH, dtype=src.dtype) % N).reshape(pad, _CH)
    srcp = _lay(jnp.concatenate([src.reshape(nch, _CH), pad_idx]))
    dstp = _lay(jnp.concatenate([dst.reshape(nch, _CH), pad_idx]))
    idx2 = jnp.stack([srcp, dstp], axis=1)
    wp1 = _lay(jnp.concatenate(
        [ew.reshape(nch, _CH), jnp.zeros((pad, _CH), f32)]))

    bm_grid = N // 1000
    BM = N // bm_grid
    nspec = pl.BlockSpec((BM, 1), lambda i: (i, 0))
    fspec = pl.BlockSpec((BM, Hh), lambda i: (i, 0))

    deg1 = _sc_deg(idx2, wp1, zeros_n, N, iters, stride)
    d1c = deg1.reshape(_NC, N, 1)

    def tc1(d0_ref, d1_ref, x_ref, w_ref, oa_ref, ob_ref):
        dinv = lax.rsqrt(1.0 + d0_ref[...] + d1_ref[...])
        hseg = jnp.dot(x_ref[...], w_ref[...], preferred_element_type=f32)
        hseg = dinv * hseg
        oa_ref[...] = hseg[:, :Hh]
        ob_ref[...] = hseg[:, Hh:]

    h1a, h1b = pl.pallas_call(
        tc1,
        grid=(bm_grid,),
        in_specs=[
            nspec, nspec,
            pl.BlockSpec((BM, Din), lambda i: (i, 0)),
            pl.BlockSpec((Din, H1), lambda i: (0, 0)),
        ],
        out_specs=[fspec, fspec],
        out_shape=[jax.ShapeDtypeStruct((N, Hh), f32)] * 2,
    )(d1c[0], d1c[1], node_attr, W1)

    acc1 = _sc_spmm(idx2, wp1, [h1a, h1b], zeros_f, N, iters, stride, Hh)

    def tc2(d0_ref, d1_ref, aa0_ref, aa1_ref, ab0_ref, ab1_ref, ha_ref,
            hb_ref, b1_ref, wm_ref, bm_ref, w2_ref, p_ref, q_ref, h2_ref):
        dinv = lax.rsqrt(1.0 + d0_ref[...] + d1_ref[...])
        suma = aa0_ref[...] + aa1_ref[...] + ha_ref[...]
        sumb = ab0_ref[...] + ab1_ref[...] + hb_ref[...]
        outa = jnp.maximum(dinv * suma + b1_ref[..., :Hh], 0.0)
        outb = jnp.maximum(dinv * sumb + b1_ref[..., Hh:], 0.0)
        out1 = jnp.concatenate([outa, outb], axis=1)
        wm = wm_ref[...]
        p_ref[...] = jnp.dot(out1, wm[:H1], preferred_element_type=f32) \
            + bm_ref[...]
        q_ref[...] = jnp.dot(out1, wm[H1:], preferred_element_type=f32)
        h2_ref[...] = jnp.dot(out1, w2_ref[...], preferred_element_type=f32)

    p, q, h2 = pl.pallas_call(
        tc2,
        grid=(bm_grid,),
        in_specs=[
            nspec, nspec, fspec, fspec, fspec, fspec, fspec, fspec,
            pl.BlockSpec((1, H1), lambda i: (0, 0)),
            pl.BlockSpec((2 * H1, 1), lambda i: (0, 0)),
            pl.BlockSpec((1, 1), lambda i: (0, 0)),
            pl.BlockSpec((H1, H2), lambda i: (0, 0)),
        ],
        out_specs=[nspec, nspec, fspec],
        out_shape=[
            jax.ShapeDtypeStruct((N, 1), f32),
            jax.ShapeDtypeStruct((N, 1), f32),
            jax.ShapeDtypeStruct((N, H2), f32),
        ],
    )(d1c[0], d1c[1], acc1[0, 0], acc1[0, 1], acc1[1, 0], acc1[1, 1],
      h1a, h1b, b1.reshape(1, H1), Wm, bm.reshape(1, 1), W2)

    wp2, deg2 = _sc_edge_mlp(idx2, p.reshape(N), q.reshape(N), zeros_n,
                             N, iters, stride, nch)
    d2c = deg2.reshape(_NC, N, 1)

    def tcd2(d0_ref, d1_ref, h2_ref, o_ref):
        dinv = lax.rsqrt(1.0 + d0_ref[...] + d1_ref[...])
        o_ref[...] = dinv * h2_ref[...]

    h2s = pl.pallas_call(
        tcd2,
        grid=(bm_grid,),
        in_specs=[nspec, nspec, fspec],
        out_specs=fspec,
        out_shape=jax.ShapeDtypeStruct((N, H2), f32),
    )(d2c[0], d2c[1], h2)

    acc2 = _sc_spmm(idx2, wp2, [h2s], zeros_f, N, iters, stride, H2)

    def tc3(d0_ref, d1_ref, a0_ref, a1_ref, h2_ref, b2_ref, o_ref):
        dinv = lax.rsqrt(1.0 + d0_ref[...] + d1_ref[...])
        s = a0_ref[...] + a1_ref[...] + h2_ref[...]
        o_ref[...] = dinv * s + b2_ref[...]

    out = pl.pallas_call(
        tc3,
        grid=(bm_grid,),
        in_specs=[
            nspec, nspec, fspec, fspec, fspec,
            pl.BlockSpec((1, H2), lambda i: (0, 0)),
        ],
        out_specs=fspec,
        out_shape=jax.ShapeDtypeStruct((N, H2), f32),
    )(d2c[0], d2c[1], acc2[0, 0], acc2[0, 1], h2s, b2.reshape(1, H2))
    return out


# speedup vs baseline: 1.1658x; 1.0035x over previous
import functools

import jax
import jax.numpy as jnp
from jax import lax
from jax.experimental import pallas as pl
from jax.experimental.pallas import tpu as pltpu
from jax.experimental.pallas import tpu_sc as plsc

_NC = 2
_NS = 16
_NW = _NC * _NS
_L = 16
_CH = 128


def _mesh():
    return plsc.VectorSubcoreMesh(core_axis_name="c", subcore_axis_name="s")


def _sc_deg(idx2, wp, zeros_n, N, iters, stride):
    @functools.partial(
        pl.kernel,
        out_type=jax.ShapeDtypeStruct((_NC, N), jnp.float32),
        mesh=_mesh(),
        compiler_params=pltpu.CompilerParams(needs_layout_passes=False),
        scratch_types=[
            pltpu.VMEM((stride, 2, _CH), jnp.int32),
            pltpu.VMEM((stride, _CH), jnp.float32),
            pltpu.VMEM_SHARED((N,), jnp.float32),
            pltpu.SemaphoreType.DMA,
        ],
    )
    def k(idx_h, wp_h, zn_h, out_h, meta_v, w_v, deg_sh, sem):
        cid = lax.axis_index("c")
        sid = lax.axis_index("s")
        wid = sid * _NC + cid
        base = wid * stride
        pltpu.sync_copy(idx_h.at[pl.ds(base, stride)], meta_v)
        pltpu.sync_copy(wp_h.at[pl.ds(base, stride)], w_v)

        @pl.when(sid == 0)
        def _():
            pltpu.sync_copy(zn_h, deg_sh)

        plsc.subcore_barrier()

        def cbody(i, carry):
            descs = []
            for b in range(4):
                c = i * 4 + b
                descs.append(pltpu.async_copy(
                    w_v.at[c], deg_sh.at[meta_v.at[c, 1]], sem, add=True))
            for d in descs:
                d.wait()
            return carry

        lax.fori_loop(0, iters // 4, cbody, 0)
        plsc.subcore_barrier()

        @pl.when(sid == 0)
        def _():
            pltpu.sync_copy(deg_sh, out_h.at[cid])

    return k(idx2, wp, zeros_n)


def _sc_spmm(idx2, wp, tables, zeros_f, N, iters, stride, D):
    nh = len(tables)
    nv = D // _L

    @functools.partial(
        pl.kernel,
        out_type=jax.ShapeDtypeStruct((nh, _NC, N, D), jnp.float32),
        mesh=_mesh(),
        compiler_params=pltpu.CompilerParams(needs_layout_passes=False),
        scratch_types=[
            pltpu.VMEM((stride, 2, _CH), jnp.int32),
            pltpu.VMEM((stride, _CH), jnp.float32),
            pltpu.VMEM((_CH, D), jnp.float32),
            pltpu.VMEM((_CH, D), jnp.float32),
            pltpu.VMEM_SHARED((N, D), jnp.float32),
            pltpu.SemaphoreType.DMA,
            pltpu.SemaphoreType.DMA,
            pltpu.SemaphoreType.DMA,
            pltpu.SemaphoreType.DMA,
        ],
    )
    def k(idx_h, wp_h, *rest):
        tabs = rest[:nh]
        zf_h = rest[nh]
        out_h = rest[nh + 1]
        (meta_v, w_v, rows0, rows1, acc_sh,
         sg0, sg1, ss0, ss1) = rest[nh + 2:]
        rows = (rows0, rows1)
        sg = (sg0, sg1)
        ss = (ss0, ss1)
        cid = lax.axis_index("c")
        sid = lax.axis_index("s")
        wid = sid * _NC + cid
        base = wid * stride
        pltpu.sync_copy(idx_h.at[pl.ds(base, stride)], meta_v)
        pltpu.sync_copy(wp_h.at[pl.ds(base, stride)], w_v)

        def scale(c, rows_v):
            def sbody(g):
                cvec = w_v[c, pl.ds(g * _L, _L)]
                for t in range(_L):
                    ce = cvec[t]
                    e = g * _L + t
                    for v in range(nv):
                        sv = pl.ds(v * _L, _L)
                        rows_v[e, sv] = rows_v[e, sv] * ce

            plsc.parallel_loop(0, _CH // _L)(sbody)

        for h in range(nh):
            tab = tabs[h]

            @pl.when(sid == 0)
            def _():
                pltpu.sync_copy(zf_h, acc_sh)

            plsc.subcore_barrier()
            def issue_gather(c, buf, sem):
                descs = []
                for q in range(4):
                    qs = pl.ds(q * (_CH // 4), _CH // 4)
                    descs.append(pltpu.async_copy(
                        tab.at[meta_v.at[c, 0, qs]], buf.at[qs], sem))
                return descs

            for d in issue_gather(0, rows[0], sg[0]):
                d.wait()

            def cbody(i, carry):
                for b in range(2):
                    c = i * 2 + b
                    nxt = jnp.minimum(c + 1, iters - 1)
                    gs = issue_gather(nxt, rows[1 - b], sg[1 - b])
                    scale(c, rows[b])
                    pltpu.sync_copy(
                        rows[b], acc_sh.at[meta_v.at[c, 1]], add=True)
                    for g in gs:
                        g.wait()
                return carry

            lax.fori_loop(0, iters // 2, cbody, 0)
            plsc.subcore_barrier()

            @pl.when(sid == 0)
            def _():
                pltpu.sync_copy(acc_sh, out_h.at[h, cid])

            plsc.subcore_barrier()

    return k(idx2, wp, *tables, zeros_f)


def _sc_edge_mlp(idx2, p, q, zeros_n, N, iters, stride, nch):
    @functools.partial(
        pl.kernel,
        out_type=[
            jax.ShapeDtypeStruct((_NW * stride, _CH), jnp.float32),
            jax.ShapeDtypeStruct((_NC, N), jnp.float32),
        ],
        mesh=_mesh(),
        compiler_params=pltpu.CompilerParams(needs_layout_passes=False),
        scratch_types=[
            pltpu.VMEM((N,), jnp.float32),
            pltpu.VMEM((N,), jnp.float32),
            pltpu.VMEM((stride, 2, _CH), jnp.int32),
            pltpu.VMEM((stride, _CH), jnp.float32),
            pltpu.VMEM_SHARED((N,), jnp.float32),
            pltpu.SemaphoreType.DMA,
        ],
    )
    def k(idx_h, p_h, q_h, zn_h, ea_h, dout_h, p_v, q_v, meta_v, ea_v,
          deg_sh, sem):
        cid = lax.axis_index("c")
        sid = lax.axis_index("s")
        wid = sid * _NC + cid
        base = wid * stride
        pltpu.sync_copy(idx_h.at[pl.ds(base, stride)], meta_v)
        pltpu.sync_copy(p_h, p_v)
        pltpu.sync_copy(q_h, q_v)

        @pl.when(sid == 0)
        def _():
            pltpu.sync_copy(zn_h, deg_sh)

        plsc.subcore_barrier()

        def cbody(i, carry):
            descs = []
            for b in range(2):
                c = i * 2 + b
                live = (wid * iters + c) < nch
                for j in range(_CH // _L):
                    sl = pl.ds(j * _L, _L)
                    ps = plsc.load_gather(p_v, [meta_v[c, 0, sl]])
                    qd = plsc.load_gather(q_v, [meta_v[c, 1, sl]])
                    ea_v[c, sl] = jnp.where(
                        live, jnp.maximum(ps + qd, 0.0), 0.0)
                descs.append(pltpu.async_copy(
                    ea_v.at[c], deg_sh.at[meta_v.at[c, 1]], sem, add=True))
            for d in descs:
                d.wait()
            return carry

        lax.fori_loop(0, iters // 2, cbody, 0)
        pltpu.sync_copy(ea_v, ea_h.at[pl.ds(base, stride)])
        plsc.subcore_barrier()

        @pl.when(sid == 0)
        def _():
            pltpu.sync_copy(deg_sh, dout_h.at[cid])

    return k(idx2, p, q, zeros_n)


def kernel(node_attr, edge_index, edge_attr, W1, b1, W2, b2, Wm, bm):
    N, Din = node_attr.shape
    E = edge_index.shape[1]
    H1 = W1.shape[1]
    H2 = W2.shape[1]
    Hh = H1 // 2
    f32 = jnp.float32

    src = edge_index[0]
    dst = edge_index[1]
    ew = edge_attr.reshape(-1)
    zeros_n = jnp.zeros((N,), f32)
    zeros_f = jnp.zeros((N, Hh), f32)

    nch = E // _CH
    iters = -(-nch // _NW)
    iters = -(-iters // 4) * 4
    stride = -(-iters // 8) * 8
    slots = iters * _NW
    pad = slots - nch

    def _lay(x2d):
        x = x2d.reshape(_NW, iters, _CH)
        x = jnp.pad(x, ((0, 0), (0, stride - iters), (0, 0)))
        return x.reshape(_NW * stride, _CH)

    pad_idx = (jnp.arange(pad * _CH, dtype=src.dtype) % N).reshape(pad, _CH)
    srcp = _lay(jnp.concatenate([src.reshape(nch, _CH), pad_idx]))
    dstp = _lay(jnp.concatenate([dst.reshape(nch, _CH), pad_idx]))
    idx2 = jnp.stack([srcp, dstp], axis=1)
    wp1 = _lay(jnp.concatenate(
        [ew.reshape(nch, _CH), jnp.zeros((pad, _CH), f32)]))

    bm_grid = N // 1000
    BM = N // bm_grid
    nspec = pl.BlockSpec((BM, 1), lambda i: (i, 0))
    fspec = pl.BlockSpec((BM, Hh), lambda i: (i, 0))

    deg1 = _sc_deg(idx2, wp1, zeros_n, N, iters, stride)
    d1c = deg1.reshape(_NC, N, 1)

    def tc1(d0_ref, d1_ref, x_ref, w_ref, oa_ref, ob_ref):
        dinv = lax.rsqrt(1.0 + d0_ref[...] + d1_ref[...])
        hseg = jnp.dot(x_ref[...], w_ref[...], preferred_element_type=f32)
        hseg = dinv * hseg
        oa_ref[...] = hseg[:, :Hh]
        ob_ref[...] = hseg[:, Hh:]

    h1a, h1b = pl.pallas_call(
        tc1,
        grid=(bm_grid,),
        in_specs=[
            nspec, nspec,
            pl.BlockSpec((BM, Din), lambda i: (i, 0)),
            pl.BlockSpec((Din, H1), lambda i: (0, 0)),
        ],
        out_specs=[fspec, fspec],
        out_shape=[jax.ShapeDtypeStruct((N, Hh), f32)] * 2,
    )(d1c[0], d1c[1], node_attr, W1)

    acc1 = _sc_spmm(idx2, wp1, [h1a, h1b], zeros_f, N, iters, stride, Hh)

    def tc2(d0_ref, d1_ref, aa0_ref, aa1_ref, ab0_ref, ab1_ref, ha_ref,
            hb_ref, b1_ref, wm_ref, bm_ref, w2_ref, p_ref, q_ref, h2_ref):
        dinv = lax.rsqrt(1.0 + d0_ref[...] + d1_ref[...])
        suma = aa0_ref[...] + aa1_ref[...] + ha_ref[...]
        sumb = ab0_ref[...] + ab1_ref[...] + hb_ref[...]
        outa = jnp.maximum(dinv * suma + b1_ref[..., :Hh], 0.0)
        outb = jnp.maximum(dinv * sumb + b1_ref[..., Hh:], 0.0)
        out1 = jnp.concatenate([outa, outb], axis=1)
        wm = wm_ref[...]
        p_ref[...] = jnp.dot(out1, wm[:H1], preferred_element_type=f32) \
            + bm_ref[...]
        q_ref[...] = jnp.dot(out1, wm[H1:], preferred_element_type=f32)
        h2_ref[...] = jnp.dot(out1, w2_ref[...], preferred_element_type=f32)

    p, q, h2 = pl.pallas_call(
        tc2,
        grid=(bm_grid,),
        in_specs=[
            nspec, nspec, fspec, fspec, fspec, fspec, fspec, fspec,
            pl.BlockSpec((1, H1), lambda i: (0, 0)),
            pl.BlockSpec((2 * H1, 1), lambda i: (0, 0)),
            pl.BlockSpec((1, 1), lambda i: (0, 0)),
            pl.BlockSpec((H1, H2), lambda i: (0, 0)),
        ],
        out_specs=[nspec, nspec, fspec],
        out_shape=[
            jax.ShapeDtypeStruct((N, 1), f32),
            jax.ShapeDtypeStruct((N, 1), f32),
            jax.ShapeDtypeStruct((N, H2), f32),
        ],
    )(d1c[0], d1c[1], acc1[0, 0], acc1[0, 1], acc1[1, 0], acc1[1, 1],
      h1a, h1b, b1.reshape(1, H1), Wm, bm.reshape(1, 1), W2)

    wp2, deg2 = _sc_edge_mlp(idx2, p.reshape(N), q.reshape(N), zeros_n,
                             N, iters, stride, nch)
    d2c = deg2.reshape(_NC, N, 1)

    def tcd2(d0_ref, d1_ref, h2_ref, o_ref):
        dinv = lax.rsqrt(1.0 + d0_ref[...] + d1_ref[...])
        o_ref[...] = dinv * h2_ref[...]

    h2s = pl.pallas_call(
        tcd2,
        grid=(bm_grid,),
        in_specs=[nspec, nspec, fspec],
        out_specs=fspec,
        out_shape=jax.ShapeDtypeStruct((N, H2), f32),
    )(d2c[0], d2c[1], h2)

    acc2 = _sc_spmm(idx2, wp2, [h2s], zeros_f, N, iters, stride, H2)

    def tc3(d0_ref, d1_ref, a0_ref, a1_ref, h2_ref, b2_ref, o_ref):
        dinv = lax.rsqrt(1.0 + d0_ref[...] + d1_ref[...])
        s = a0_ref[...] + a1_ref[...] + h2_ref[...]
        o_ref[...] = dinv * s + b2_ref[...]

    out = pl.pallas_call(
        tc3,
        grid=(bm_grid,),
        in_specs=[
            nspec, nspec, fspec, fspec, fspec,
            pl.BlockSpec((1, H2), lambda i: (0, 0)),
        ],
        out_specs=fspec,
        out_shape=jax.ShapeDtypeStruct((N, H2), f32),
    )(d2c[0], d2c[1], acc2[0, 0], acc2[0, 1], h2s, b2.reshape(1, H2))
    return out
